# baseline (speedup 1.0000x reference)
import functools

import jax
import jax.numpy as jnp
from jax import lax
from jax.experimental import pallas as pl
from jax.experimental.pallas import tpu as pltpu
from jax.experimental.pallas import tpu_sc as plsc

N_ATOMS = 10000
N_EDGES = 320000
NRAD = 16
NSPEC = 100
NSPH = 9
RCUT = 5.0

N_PAD = 10016
E_PAD = 323584
EDGES_PER_TILE = E_PAD // 32
CHUNK = 64
NCHUNK = EDGES_PER_TILE // CHUNK
NPAIR = NCHUNK // 2
ROWS_PER_TILE = N_PAD // 16
F = NSPH * NRAD
BE = 2048


def _t16(a):
    eye = jnp.eye(16, dtype=jnp.float32)
    return lax.dot_general(a, eye, (((0,), (0,)), ((), ())),
                           preferred_element_type=jnp.float32)


def _edge_feat_body(dispT_ref, phi_ref, sph_ref):
    d = dispT_ref[...]
    x, y, z = d[0:1, :], d[1:2, :], d[2:3, :]
    r2 = x * x + y * y + z * z
    r = jnp.sqrt(r2 + 1e-12)
    xh, yh, zh = x / r, y / r, z / r
    cut = jnp.where(r < RCUT, 0.5 * (jnp.cos(jnp.pi * r / RCUT) + 1.0), 0.0)
    k = (lax.broadcasted_iota(jnp.int32, (16, BE), 0) + 1).astype(jnp.float32)
    xarg = k * (jnp.pi / RCUT) * r
    phiT = (jnp.sin(xarg) / xarg) * cut
    zeros7 = jnp.zeros((7, BE), jnp.float32)
    sphT = jnp.concatenate(
        [jnp.ones_like(x), yh, zh, xh, xh * yh, yh * zh,
         3.0 * zh * zh - 1.0, xh * zh, xh * xh - yh * yh, zeros7], axis=0)
    phi_ref[...] = _t16(phiT)
    sph_ref[...] = _t16(sphT)


def _edge_features(dispT):
    grid = E_PAD // BE
    return pl.pallas_call(
        _edge_feat_body,
        grid=(grid,),
        in_specs=[pl.BlockSpec((3, BE), lambda i: (0, i))],
        out_specs=[pl.BlockSpec((BE, 16), lambda i: (i, 0)),
                   pl.BlockSpec((BE, 16), lambda i: (i, 0))],
        out_shape=[jax.ShapeDtypeStruct((E_PAD, 16), jnp.float32),
                   jax.ShapeDtypeStruct((E_PAD, 16), jnp.float32)],
    )(dispT)


def _node_embed_body(z_ref, emb_ref, eW_ref, eb_ref, ne_ref, y0_ref):
    z = z_ref[...]
    iot = lax.broadcasted_iota(jnp.int32, (N_PAD, NSPEC), 1)
    oh = (z == iot).astype(jnp.float32)
    ne = jnp.dot(oh, emb_ref[...], preferred_element_type=jnp.float32)
    ne_ref[...] = ne
    y0_ref[...] = jnp.dot(ne, eW_ref[...],
                          preferred_element_type=jnp.float32) + eb_ref[...]


def _node_embed(z_pad, emb, eW, eb):
    return pl.pallas_call(
        _node_embed_body,
        out_shape=[jax.ShapeDtypeStruct((N_PAD, 16), jnp.float32),
                   jax.ShapeDtypeStruct((N_PAD, 16), jnp.float32)],
    )(z_pad, emb, eW, eb)


def _dense_body(acc_ref, W_ref, bias_ref, invn_ref, extra_ref, y_ref):
    a = acc_ref[0:N_PAD, :] + acc_ref[N_PAD:2 * N_PAD, :]
    y1 = a * invn_ref[0, 0]
    h = jnp.dot(y1, W_ref[...], preferred_element_type=jnp.float32)
    y_ref[...] = h + y1 + bias_ref[...] + extra_ref[...]


def _dense(acc, W144, bias_row, invn, extra):
    return pl.pallas_call(
        _dense_body,
        out_shape=jax.ShapeDtypeStruct((N_PAD, F), jnp.float32),
    )(acc, W144, bias_row, invn, extra)


def _lane_bcast(v, s):
    idx = jnp.full((16, 1), s, jnp.int32)
    dn = lax.GatherDimensionNumbers(offset_dims=(), collapsed_slice_dims=(0,),
                                    start_index_map=(0,))
    return lax.gather(v, idx, dn, (1,),
                      mode=lax.GatherScatterMode.PROMISE_IN_BOUNDS)


def _sc_scratch():
    return [
        pltpu.VMEM_SHARED((N_PAD, F), jnp.float32),
        pltpu.VMEM((2, CHUNK), jnp.int32),
        pltpu.VMEM((2, CHUNK), jnp.int32),
        pltpu.VMEM((2, CHUNK), jnp.int32),
        pltpu.VMEM((2, CHUNK, 16), jnp.float32),
        pltpu.VMEM((2, CHUNK, 16), jnp.float32),
        pltpu.VMEM((F,), jnp.float32),
        pltpu.VMEM((2, CHUNK, F), jnp.float32),
        pltpu.SemaphoreType.DMA, pltpu.SemaphoreType.DMA,
        pltpu.SemaphoreType.DMA, pltpu.SemaphoreType.DMA,
        pltpu.SemaphoreType.DMA, pltpu.SemaphoreType.DMA,
    ]


def _sc_pass(gather_issue, idxi_hbm, idxj_hbm, phi_hbm, sph_hbm, b_hbm,
             zeros_hbm, out_hbm, acc, idxi_v, idxj_v, sidx_v, phi_v, sph_v,
             b_v, m_v, sems, per_edge):
    c_ax = lax.axis_index("c")
    s_ax = lax.axis_index("s")
    wid = s_ax * 2 + c_ax
    isem, gsem, ssem = sems[0:2], sems[2:4], sems[4:6]
    pltpu.sync_copy(zeros_hbm,
                    acc.at[pl.ds(s_ax * ROWS_PER_TILE, ROWS_PER_TILE)])
    pltpu.sync_copy(b_hbm, b_v)
    bvs = [b_v[pl.ds(k * 16, 16)] for k in range(NSPH)]
    plsc.subcore_barrier()
    tile_base = wid * EDGES_PER_TILE

    def in_issue(c, b):
        base = tile_base + c * CHUNK
        pltpu.async_copy(idxi_hbm.at[pl.ds(base, CHUNK)], idxi_v.at[b],
                         isem[b])
        pltpu.async_copy(idxj_hbm.at[pl.ds(base, CHUNK)], idxj_v.at[b],
                         isem[b])
        pltpu.async_copy(phi_hbm.at[pl.ds(base, CHUNK)], phi_v.at[b], isem[b])
        pltpu.async_copy(sph_hbm.at[pl.ds(base, CHUNK)], sph_v.at[b], isem[b])

    def in_wait(b):
        pltpu.make_async_copy(idxi_hbm.at[pl.ds(0, CHUNK)], idxi_v.at[b],
                              isem[b]).wait()
        pltpu.make_async_copy(idxj_hbm.at[pl.ds(0, CHUNK)], idxj_v.at[b],
                              isem[b]).wait()
        pltpu.make_async_copy(phi_hbm.at[pl.ds(0, CHUNK)], phi_v.at[b],
                              isem[b]).wait()
        pltpu.make_async_copy(sph_hbm.at[pl.ds(0, CHUNK)], sph_v.at[b],
                              isem[b]).wait()

    def g_issue(b):
        gather_issue(b, idxj_v.at[b], m_v.at[b], gsem[b], issue=True)

    def g_wait(b):
        gather_issue(b, idxj_v.at[b], m_v.at[b], gsem[b], issue=False)

    def s_issue(b):
        for t in range(CHUNK // 16):
            sidx_v[b, pl.ds(t * 16, 16)] = idxi_v[b, pl.ds(t * 16, 16)]
        pltpu.async_copy(m_v.at[b], acc.at[sidx_v.at[b]], ssem[b], add=True)

    def s_wait(b):
        pltpu.make_async_copy(m_v.at[b], acc.at[sidx_v.at[b]],
                              ssem[b]).wait()

    def cmp(b):
        def _body(t, carry):
            for q in range(4):
                per_edge(t * 4 + q, b, phi_v, sph_v, bvs, m_v)
            return carry
        lax.fori_loop(0, CHUNK // 4, _body, 0)

    def step(c, b, first, has_next, has_next2):
        g_wait(b)
        if not first:
            s_wait(1 - b)
        if has_next:
            in_wait(1 - b)
            g_issue(1 - b)
        cmp(b)
        s_issue(b)
        if has_next2:
            in_issue(c + 2, b)

    in_issue(0, 0)
    in_wait(0)
    g_issue(0)
    in_issue(1, 1)
    step(0, 0, True, True, True)
    step(1, 1, False, True, True)

    def pair(g, carry):
        step(g * 2, 0, False, True, True)
        step(g * 2 + 1, 1, False, True, True)
        return carry
    lax.fori_loop(1, NPAIR - 1, pair, 0)
    step(NCHUNK - 2, 0, False, True, False)
    step(NCHUNK - 1, 1, False, False, False)
    s_wait(1)

    plsc.subcore_barrier()
    row0 = s_ax * ROWS_PER_TILE
    pltpu.sync_copy(acc.at[pl.ds(row0, ROWS_PER_TILE)],
                    out_hbm.at[pl.ds(c_ax * N_PAD + row0, ROWS_PER_TILE)])


@functools.lru_cache(maxsize=None)
def _make_sc_pass1():
    mesh = plsc.VectorSubcoreMesh(core_axis_name="c", subcore_axis_name="s")

    @functools.partial(
        pl.kernel, mesh=mesh,
        out_type=jax.ShapeDtypeStruct((2 * N_PAD, F), jnp.float32),
        scratch_types=_sc_scratch() + [pltpu.VMEM((2, CHUNK, 16),
                                                  jnp.float32)],
        compiler_params=pltpu.CompilerParams(use_tc_tiling_on_sc=False),
    )
    def _sc_pass1(idxi_hbm, idxj_hbm, phi_hbm, sph_hbm, ne_hbm, b1_hbm,
                  zeros_hbm, out_hbm, acc, idxi_v, idxj_v, sidx_v, phi_v,
                  sph_v, b_v, m_v, is0, is1, gs0, gs1, ss0, ss1, ne_v):
        def gather_rows(b, idxj_slot, m_slot, sem, issue):
            cp = pltpu.make_async_copy(ne_hbm.at[idxj_slot], ne_v.at[b], sem)
            if issue:
                cp.start()
            else:
                cp.wait()

        def per_edge(e, b, phi_ref, sph_ref, bvs, m_ref):
            phi = phi_ref[b, e, :]
            sphv = sph_ref[b, e, :]
            ne = ne_v[b, e, :]
            ye = phi * ne
            u = phi * ye
            for k in range(NSPH):
                m = _lane_bcast(sphv, k) * u + bvs[k] * ye
                m_ref[b, e, pl.ds(k * 16, 16)] = m

        _sc_pass(gather_rows, idxi_hbm, idxj_hbm, phi_hbm, sph_hbm, b1_hbm,
                 zeros_hbm, out_hbm, acc, idxi_v, idxj_v, sidx_v, phi_v,
                 sph_v, b_v, m_v, [is0, is1, gs0, gs1, ss0, ss1], per_edge)

    return _sc_pass1


@functools.lru_cache(maxsize=None)
def _make_sc_pass2():
    mesh = plsc.VectorSubcoreMesh(core_axis_name="c", subcore_axis_name="s")

    @functools.partial(
        pl.kernel, mesh=mesh,
        out_type=jax.ShapeDtypeStruct((2 * N_PAD, F), jnp.float32),
        scratch_types=_sc_scratch(),
        compiler_params=pltpu.CompilerParams(use_tc_tiling_on_sc=False),
    )
    def _sc_pass2(idxi_hbm, idxj_hbm, phi_hbm, sph_hbm, y_hbm, b2_hbm,
                  zeros_hbm, out_hbm, acc, idxi_v, idxj_v, sidx_v, phi_v,
                  sph_v, b_v, m_v, is0, is1, gs0, gs1, ss0, ss1):
        def gather_rows(b, idxj_slot, m_slot, sem, issue):
            cp = pltpu.make_async_copy(y_hbm.at[idxj_slot], m_slot, sem)
            if issue:
                cp.start()
            else:
                cp.wait()

        def per_edge(e, b, phi_ref, sph_ref, bvs, m_ref):
            phi = phi_ref[b, e, :]
            sphv = sph_ref[b, e, :]
            for k in range(NSPH):
                t = _lane_bcast(sphv, k) * phi + bvs[k]
                m_ref[b, e, pl.ds(k * 16, 16)] = (
                    t * m_ref[b, e, pl.ds(k * 16, 16)])

        _sc_pass(gather_rows, idxi_hbm, idxj_hbm, phi_hbm, sph_hbm, b2_hbm,
                 zeros_hbm, out_hbm, acc, idxi_v, idxj_v, sidx_v, phi_v,
                 sph_v, b_v, m_v, [is0, is1, gs0, gs1, ss0, ss1], per_edge)

    return _sc_pass2


def kernel(atomic_numbers, neighbour_indices, neighbour_displacements,
           embedding_table, embed_W, embed_b, dense1_W, dense1_b, dense2_W,
           dense2_b, basis_b1, basis_b2, neighbour_normalization):
    f32 = jnp.float32
    pad_e = E_PAD - N_EDGES
    dummy = (N_ATOMS + (jnp.arange(pad_e, dtype=jnp.int32) % 16)).astype(jnp.int32)
    idx = neighbour_indices.astype(jnp.int32)
    idx_i = jnp.concatenate([idx[:, 0], dummy])
    idx_j = jnp.concatenate([idx[:, 1], dummy])
    disp_pad = jnp.concatenate(
        [neighbour_displacements.astype(f32),
         jnp.tile(jnp.asarray([[2.0 * RCUT, 0.0, 0.0]], f32), (pad_e, 1))])
    dispT = disp_pad.T

    z_pad = jnp.concatenate(
        [atomic_numbers.astype(jnp.int32),
         jnp.zeros((N_PAD - N_ATOMS,), jnp.int32)]).reshape(N_PAD, 1)

    eye9 = jnp.eye(NSPH, dtype=f32)
    W144_1 = jnp.kron(eye9, dense1_W.astype(f32))
    W144_2 = jnp.kron(eye9, dense2_W.astype(f32))
    bias1 = jnp.concatenate([dense1_b.astype(f32),
                             jnp.zeros((F - 16,), f32)]).reshape(1, F)
    bias2 = jnp.concatenate([dense2_b.astype(f32),
                             jnp.zeros((F - 16,), f32)]).reshape(1, F)
    invn = (1.0 / neighbour_normalization.astype(f32)).reshape(1, 1)
    zeros_stripe = jnp.zeros((ROWS_PER_TILE, F), f32)
    zero_extra = jnp.zeros((1, F), f32)

    phi_arr, sph_arr = _edge_features(dispT)
    ne, y0 = _node_embed(z_pad, embedding_table.astype(f32),
                         embed_W.astype(f32), embed_b.astype(f32).reshape(1, 16))

    acc1 = _make_sc_pass1()(idx_i, idx_j, phi_arr, sph_arr, ne,
                            basis_b1.astype(f32).reshape(F), zeros_stripe)
    y_mid = _dense(acc1, W144_1, bias1, invn, zero_extra)
    acc2 = _make_sc_pass2()(idx_i, idx_j, phi_arr, sph_arr, y_mid,
                            basis_b2.astype(f32).reshape(F), zeros_stripe)
    y0_ext = jnp.concatenate([y0, jnp.zeros((N_PAD, F - 16), f32)], axis=1)
    y_out = _dense(acc2, W144_2, bias2, invn, y0_ext)

    return y_out[:N_ATOMS].reshape(N_ATOMS, NSPH, NRAD)


# speedup vs baseline: 1.1529x; 1.1529x over previous
import functools

import jax
import jax.numpy as jnp
from jax import lax
from jax.experimental import pallas as pl
from jax.experimental.pallas import tpu as pltpu
from jax.experimental.pallas import tpu_sc as plsc

N_ATOMS = 10000
N_EDGES = 320000
NRAD = 16
NSPEC = 100
NSPH = 9
RCUT = 5.0

N_PAD = 10016
E_PAD = 323584
EDGES_PER_TILE = E_PAD // 32
CHUNK = 64
NCHUNK = EDGES_PER_TILE // CHUNK
NPAIR = NCHUNK // 2
ROWS_PER_TILE = N_PAD // 16
F = NSPH * NRAD
BE = 2048


def _t16(a):
    eye = jnp.eye(16, dtype=jnp.float32)
    return lax.dot_general(a, eye, (((0,), (0,)), ((), ())),
                           preferred_element_type=jnp.float32)


def _edge_feat_body(dispT_ref, phi_ref, sph_ref):
    d = dispT_ref[...]
    x, y, z = d[0:1, :], d[1:2, :], d[2:3, :]
    r2 = x * x + y * y + z * z
    r = jnp.sqrt(r2 + 1e-12)
    xh, yh, zh = x / r, y / r, z / r
    cut = jnp.where(r < RCUT, 0.5 * (jnp.cos(jnp.pi * r / RCUT) + 1.0), 0.0)
    k = (lax.broadcasted_iota(jnp.int32, (16, BE), 0) + 1).astype(jnp.float32)
    xarg = k * (jnp.pi / RCUT) * r
    phiT = (jnp.sin(xarg) / xarg) * cut
    zeros7 = jnp.zeros((7, BE), jnp.float32)
    sphT = jnp.concatenate(
        [jnp.ones_like(x), yh, zh, xh, xh * yh, yh * zh,
         3.0 * zh * zh - 1.0, xh * zh, xh * xh - yh * yh, zeros7], axis=0)
    phi_ref[...] = _t16(phiT)
    sph_ref[...] = _t16(sphT)


def _edge_features(dispT):
    grid = E_PAD // BE
    return pl.pallas_call(
        _edge_feat_body,
        grid=(grid,),
        in_specs=[pl.BlockSpec((3, BE), lambda i: (0, i))],
        out_specs=[pl.BlockSpec((BE, 16), lambda i: (i, 0)),
                   pl.BlockSpec((BE, 16), lambda i: (i, 0))],
        out_shape=[jax.ShapeDtypeStruct((E_PAD, 16), jnp.float32),
                   jax.ShapeDtypeStruct((E_PAD, 16), jnp.float32)],
    )(dispT)


def _node_embed_body(z_ref, emb_ref, eW_ref, eb_ref, ne_ref, y0_ref):
    z = z_ref[...]
    iot = lax.broadcasted_iota(jnp.int32, (N_PAD, NSPEC), 1)
    oh = (z == iot).astype(jnp.float32)
    ne = jnp.dot(oh, emb_ref[...], preferred_element_type=jnp.float32)
    ne_ref[...] = ne
    y0_ref[...] = jnp.dot(ne, eW_ref[...],
                          preferred_element_type=jnp.float32) + eb_ref[...]


def _node_embed(z_pad, emb, eW, eb):
    return pl.pallas_call(
        _node_embed_body,
        out_shape=[jax.ShapeDtypeStruct((N_PAD, 16), jnp.float32),
                   jax.ShapeDtypeStruct((N_PAD, 16), jnp.float32)],
    )(z_pad, emb, eW, eb)


def _dense_body(acc_ref, W_ref, bias_ref, invn_ref, extra_ref, y_ref):
    a = acc_ref[0:N_PAD, :] + acc_ref[N_PAD:2 * N_PAD, :]
    y1 = a * invn_ref[0, 0]
    h = jnp.dot(y1, W_ref[...], preferred_element_type=jnp.float32)
    y_ref[...] = h + y1 + bias_ref[...] + extra_ref[...]


def _dense(acc, W144, bias_row, invn, extra):
    return pl.pallas_call(
        _dense_body,
        out_shape=jax.ShapeDtypeStruct((N_PAD, F), jnp.float32),
    )(acc, W144, bias_row, invn, extra)


def _lane_bcast(v, s):
    idx = jnp.full((16, 1), s, jnp.int32)
    dn = lax.GatherDimensionNumbers(offset_dims=(), collapsed_slice_dims=(0,),
                                    start_index_map=(0,))
    return lax.gather(v, idx, dn, (1,),
                      mode=lax.GatherScatterMode.PROMISE_IN_BOUNDS)


def _sc_scratch():
    return [
        pltpu.VMEM_SHARED((N_PAD, F), jnp.float32),
        pltpu.VMEM((2, CHUNK), jnp.int32),
        pltpu.VMEM((2, CHUNK), jnp.int32),
        pltpu.VMEM((2, CHUNK), jnp.int32),
        pltpu.VMEM((2, CHUNK, 16), jnp.float32),
        pltpu.VMEM((2, CHUNK, 16), jnp.float32),
        pltpu.VMEM((F,), jnp.float32),
        pltpu.VMEM((2, CHUNK, F), jnp.float32),
        pltpu.SemaphoreType.DMA, pltpu.SemaphoreType.DMA,
        pltpu.SemaphoreType.DMA, pltpu.SemaphoreType.DMA,
        pltpu.SemaphoreType.DMA, pltpu.SemaphoreType.DMA,
    ]


def _sc_pass(gather_issue, idxi_hbm, idxj_hbm, phi_hbm, sph_hbm, b_hbm,
             zeros_hbm, out_hbm, acc, idxi_v, idxj_v, sidx_v, phi_v, sph_v,
             b_v, m_v, sems, per_edge):
    c_ax = lax.axis_index("c")
    s_ax = lax.axis_index("s")
    wid = s_ax * 2 + c_ax
    isem, gsem, ssem = sems[0:2], sems[2:4], sems[4:6]
    pltpu.sync_copy(zeros_hbm,
                    acc.at[pl.ds(s_ax * ROWS_PER_TILE, ROWS_PER_TILE)])
    pltpu.sync_copy(b_hbm, b_v)
    bvs = [b_v[pl.ds(k * 16, 16)] for k in range(NSPH)]
    plsc.subcore_barrier()
    tile_base = wid * EDGES_PER_TILE

    def in_issue(c, b):
        base = tile_base + c * CHUNK
        pltpu.async_copy(idxi_hbm.at[pl.ds(base, CHUNK)], idxi_v.at[b],
                         isem[b])
        pltpu.async_copy(idxj_hbm.at[pl.ds(base, CHUNK)], idxj_v.at[b],
                         isem[b])
        pltpu.async_copy(phi_hbm.at[pl.ds(base, CHUNK)], phi_v.at[b], isem[b])
        pltpu.async_copy(sph_hbm.at[pl.ds(base, CHUNK)], sph_v.at[b], isem[b])

    def in_wait(b):
        pltpu.make_async_copy(idxi_hbm.at[pl.ds(0, CHUNK)], idxi_v.at[b],
                              isem[b]).wait()
        pltpu.make_async_copy(idxj_hbm.at[pl.ds(0, CHUNK)], idxj_v.at[b],
                              isem[b]).wait()
        pltpu.make_async_copy(phi_hbm.at[pl.ds(0, CHUNK)], phi_v.at[b],
                              isem[b]).wait()
        pltpu.make_async_copy(sph_hbm.at[pl.ds(0, CHUNK)], sph_v.at[b],
                              isem[b]).wait()

    def g_issue(b):
        gather_issue(b, idxj_v.at[b], m_v.at[b], gsem[b], issue=True)

    def g_wait(b):
        gather_issue(b, idxj_v.at[b], m_v.at[b], gsem[b], issue=False)

    def s_issue(b):
        for t in range(CHUNK // 16):
            sidx_v[b, pl.ds(t * 16, 16)] = idxi_v[b, pl.ds(t * 16, 16)]
        pltpu.async_copy(m_v.at[b], acc.at[sidx_v.at[b]], ssem[b], add=True)

    def s_wait(b):
        pltpu.make_async_copy(m_v.at[b], acc.at[sidx_v.at[b]],
                              ssem[b]).wait()

    def cmp(b):
        def _body(e, carry):
            per_edge(e, b, phi_v, sph_v, bvs, m_v)
            return carry
        lax.fori_loop(0, CHUNK, _body, 0)

    def step(c, b, first, has_next, has_next2):
        g_wait(b)
        if not first:
            s_wait(1 - b)
        if has_next:
            in_wait(1 - b)
            g_issue(1 - b)
        cmp(b)
        s_issue(b)
        if has_next2:
            in_issue(c + 2, b)

    in_issue(0, 0)
    in_wait(0)
    g_issue(0)
    in_issue(1, 1)
    step(0, 0, True, True, True)
    step(1, 1, False, True, True)

    def pair(g, carry):
        step(g * 2, 0, False, True, True)
        step(g * 2 + 1, 1, False, True, True)
        return carry
    lax.fori_loop(1, NPAIR - 1, pair, 0)
    step(NCHUNK - 2, 0, False, True, False)
    step(NCHUNK - 1, 1, False, False, False)
    s_wait(1)

    plsc.subcore_barrier()
    row0 = s_ax * ROWS_PER_TILE
    pltpu.sync_copy(acc.at[pl.ds(row0, ROWS_PER_TILE)],
                    out_hbm.at[pl.ds(c_ax * N_PAD + row0, ROWS_PER_TILE)])


@functools.lru_cache(maxsize=None)
def _make_sc_pass1():
    mesh = plsc.VectorSubcoreMesh(core_axis_name="c", subcore_axis_name="s")

    @functools.partial(
        pl.kernel, mesh=mesh,
        out_type=jax.ShapeDtypeStruct((2 * N_PAD, F), jnp.float32),
        scratch_types=_sc_scratch() + [pltpu.VMEM((2, CHUNK, 16),
                                                  jnp.float32)],
        compiler_params=pltpu.CompilerParams(use_tc_tiling_on_sc=False),
    )
    def _sc_pass1(idxi_hbm, idxj_hbm, phi_hbm, sph_hbm, ne_hbm, b1_hbm,
                  zeros_hbm, out_hbm, acc, idxi_v, idxj_v, sidx_v, phi_v,
                  sph_v, b_v, m_v, is0, is1, gs0, gs1, ss0, ss1, ne_v):
        def gather_rows(b, idxj_slot, m_slot, sem, issue):
            cp = pltpu.make_async_copy(ne_hbm.at[idxj_slot], ne_v.at[b], sem)
            if issue:
                cp.start()
            else:
                cp.wait()

        def per_edge(e, b, phi_ref, sph_ref, bvs, m_ref):
            phi = phi_ref[b, e, :]
            sphv = sph_ref[b, e, :]
            ne = ne_v[b, e, :]
            ye = phi * ne
            u = phi * ye
            for k in range(NSPH):
                m = _lane_bcast(sphv, k) * u + bvs[k] * ye
                m_ref[b, e, pl.ds(k * 16, 16)] = m

        _sc_pass(gather_rows, idxi_hbm, idxj_hbm, phi_hbm, sph_hbm, b1_hbm,
                 zeros_hbm, out_hbm, acc, idxi_v, idxj_v, sidx_v, phi_v,
                 sph_v, b_v, m_v, [is0, is1, gs0, gs1, ss0, ss1], per_edge)

    return _sc_pass1


@functools.lru_cache(maxsize=None)
def _make_sc_pass2():
    mesh = plsc.VectorSubcoreMesh(core_axis_name="c", subcore_axis_name="s")

    @functools.partial(
        pl.kernel, mesh=mesh,
        out_type=jax.ShapeDtypeStruct((2 * N_PAD, F), jnp.float32),
        scratch_types=_sc_scratch(),
        compiler_params=pltpu.CompilerParams(use_tc_tiling_on_sc=False),
    )
    def _sc_pass2(idxi_hbm, idxj_hbm, phi_hbm, sph_hbm, y_hbm, b2_hbm,
                  zeros_hbm, out_hbm, acc, idxi_v, idxj_v, sidx_v, phi_v,
                  sph_v, b_v, m_v, is0, is1, gs0, gs1, ss0, ss1):
        def gather_rows(b, idxj_slot, m_slot, sem, issue):
            cp = pltpu.make_async_copy(y_hbm.at[idxj_slot], m_slot, sem)
            if issue:
                cp.start()
            else:
                cp.wait()

        def per_edge(e, b, phi_ref, sph_ref, bvs, m_ref):
            phi = phi_ref[b, e, :]
            sphv = sph_ref[b, e, :]
            for k in range(NSPH):
                t = _lane_bcast(sphv, k) * phi + bvs[k]
                m_ref[b, e, pl.ds(k * 16, 16)] = (
                    t * m_ref[b, e, pl.ds(k * 16, 16)])

        _sc_pass(gather_rows, idxi_hbm, idxj_hbm, phi_hbm, sph_hbm, b2_hbm,
                 zeros_hbm, out_hbm, acc, idxi_v, idxj_v, sidx_v, phi_v,
                 sph_v, b_v, m_v, [is0, is1, gs0, gs1, ss0, ss1], per_edge)

    return _sc_pass2


def kernel(atomic_numbers, neighbour_indices, neighbour_displacements,
           embedding_table, embed_W, embed_b, dense1_W, dense1_b, dense2_W,
           dense2_b, basis_b1, basis_b2, neighbour_normalization):
    f32 = jnp.float32
    pad_e = E_PAD - N_EDGES
    dummy = (N_ATOMS + (jnp.arange(pad_e, dtype=jnp.int32) % 16)).astype(jnp.int32)
    idx = neighbour_indices.astype(jnp.int32)
    idx_i = jnp.concatenate([idx[:, 0], dummy])
    idx_j = jnp.concatenate([idx[:, 1], dummy])
    disp_pad = jnp.concatenate(
        [neighbour_displacements.astype(f32),
         jnp.tile(jnp.asarray([[2.0 * RCUT, 0.0, 0.0]], f32), (pad_e, 1))])
    dispT = disp_pad.T

    z_pad = jnp.concatenate(
        [atomic_numbers.astype(jnp.int32),
         jnp.zeros((N_PAD - N_ATOMS,), jnp.int32)]).reshape(N_PAD, 1)

    eye9 = jnp.eye(NSPH, dtype=f32)
    W144_1 = jnp.kron(eye9, dense1_W.astype(f32))
    W144_2 = jnp.kron(eye9, dense2_W.astype(f32))
    bias1 = jnp.concatenate([dense1_b.astype(f32),
                             jnp.zeros((F - 16,), f32)]).reshape(1, F)
    bias2 = jnp.concatenate([dense2_b.astype(f32),
                             jnp.zeros((F - 16,), f32)]).reshape(1, F)
    invn = (1.0 / neighbour_normalization.astype(f32)).reshape(1, 1)
    zeros_stripe = jnp.zeros((ROWS_PER_TILE, F), f32)
    zero_extra = jnp.zeros((1, F), f32)

    phi_arr, sph_arr = _edge_features(dispT)
    ne, y0 = _node_embed(z_pad, embedding_table.astype(f32),
                         embed_W.astype(f32), embed_b.astype(f32).reshape(1, 16))

    acc1 = _make_sc_pass1()(idx_i, idx_j, phi_arr, sph_arr, ne,
                            basis_b1.astype(f32).reshape(F), zeros_stripe)
    y_mid = _dense(acc1, W144_1, bias1, invn, zero_extra)
    acc2 = _make_sc_pass2()(idx_i, idx_j, phi_arr, sph_arr, y_mid,
                            basis_b2.astype(f32).reshape(F), zeros_stripe)
    y0_ext = jnp.concatenate([y0, jnp.zeros((N_PAD, F - 16), f32)], axis=1)
    y_out = _dense(acc2, W144_2, bias2, invn, y0_ext)

    return y_out[:N_ATOMS].reshape(N_ATOMS, NSPH, NRAD)


# speedup vs baseline: 1.3266x; 1.1506x over previous
import functools

import jax
import jax.numpy as jnp
from jax import lax
from jax.experimental import pallas as pl
from jax.experimental.pallas import tpu as pltpu
from jax.experimental.pallas import tpu_sc as plsc

N_ATOMS = 10000
N_EDGES = 320000
NRAD = 16
NSPEC = 100
NSPH = 9
RCUT = 5.0

N_PAD = 10016
E_PAD = 323584
EDGES_PER_TILE = E_PAD // 32
CHUNK = 64
NCHUNK = EDGES_PER_TILE // CHUNK
NPAIR = NCHUNK // 2
ROWS_PER_TILE = N_PAD // 16
F = NSPH * NRAD
BE = 2048


def _t128(a):
    eye = jnp.eye(128, dtype=jnp.float32)
    return lax.dot_general(a, eye, (((0,), (0,)), ((), ())),
                           preferred_element_type=jnp.float32)


def _edge_feat_body(disp8_ref, phi_ref, sph_ref):
    B8 = BE // 8
    d = disp8_ref[...]
    k = (lax.broadcasted_iota(jnp.int32, (16, B8), 0) + 1).astype(jnp.float32)
    phis, sphs = [], []
    for a in range(8):
        x = d[3 * a:3 * a + 1, :]
        y = d[3 * a + 1:3 * a + 2, :]
        z = d[3 * a + 2:3 * a + 3, :]
        r2 = x * x + y * y + z * z
        r = jnp.sqrt(r2 + 1e-12)
        xh, yh, zh = x / r, y / r, z / r
        cut = jnp.where(r < RCUT,
                        0.5 * (jnp.cos(jnp.pi * r / RCUT) + 1.0), 0.0)
        xarg = k * (jnp.pi / RCUT) * r
        phis.append((jnp.sin(xarg) / xarg) * cut)
        zeros7 = jnp.zeros((7, B8), jnp.float32)
        sphs.append(jnp.concatenate(
            [jnp.ones_like(x), yh, zh, xh, xh * yh, yh * zh,
             3.0 * zh * zh - 1.0, xh * zh, xh * xh - yh * yh, zeros7],
            axis=0))
    phi_ref[...] = _t128(jnp.concatenate(phis, axis=0))
    sph_ref[...] = _t128(jnp.concatenate(sphs, axis=0))


def _edge_features(disp8):
    grid = E_PAD // BE
    return pl.pallas_call(
        _edge_feat_body,
        grid=(grid,),
        in_specs=[pl.BlockSpec((24, BE // 8), lambda i: (0, i))],
        out_specs=[pl.BlockSpec((BE // 8, 128), lambda i: (i, 0)),
                   pl.BlockSpec((BE // 8, 128), lambda i: (i, 0))],
        out_shape=[jax.ShapeDtypeStruct((E_PAD // 8, 128), jnp.float32),
                   jax.ShapeDtypeStruct((E_PAD // 8, 128), jnp.float32)],
    )(disp8)


def _node_embed_body(z_ref, emb_ref, eW_ref, eb_ref, ne_ref, y0_ref):
    z = z_ref[...]
    iot = lax.broadcasted_iota(jnp.int32, (N_PAD, NSPEC), 1)
    oh = (z == iot).astype(jnp.float32)
    ne = jnp.dot(oh, emb_ref[...], preferred_element_type=jnp.float32)
    ne_ref[...] = ne
    y0_ref[...] = jnp.dot(ne, eW_ref[...],
                          preferred_element_type=jnp.float32) + eb_ref[...]


def _node_embed(z_pad, emb, eW, eb):
    return pl.pallas_call(
        _node_embed_body,
        out_shape=[jax.ShapeDtypeStruct((N_PAD, 16), jnp.float32),
                   jax.ShapeDtypeStruct((N_PAD, 16), jnp.float32)],
    )(z_pad, emb, eW, eb)


def _dense_body(acc_ref, W_ref, bias_ref, invn_ref, extra_ref, y_ref):
    a = acc_ref[0:N_PAD, :] + acc_ref[N_PAD:2 * N_PAD, :]
    y1 = a * invn_ref[0, 0]
    h = jnp.dot(y1, W_ref[...], preferred_element_type=jnp.float32)
    y_ref[...] = h + y1 + bias_ref[...] + extra_ref[...]


def _dense(acc, W144, bias_row, invn, extra):
    return pl.pallas_call(
        _dense_body,
        out_shape=jax.ShapeDtypeStruct((N_PAD, F), jnp.float32),
    )(acc, W144, bias_row, invn, extra)


def _lane_bcast(v, s):
    idx = jnp.full((16, 1), s, jnp.int32)
    dn = lax.GatherDimensionNumbers(offset_dims=(), collapsed_slice_dims=(0,),
                                    start_index_map=(0,))
    return lax.gather(v, idx, dn, (1,),
                      mode=lax.GatherScatterMode.PROMISE_IN_BOUNDS)


def _sc_scratch():
    return [
        pltpu.VMEM_SHARED((N_PAD, F), jnp.float32),
        pltpu.VMEM((2, CHUNK), jnp.int32),
        pltpu.VMEM((2, CHUNK), jnp.int32),
        pltpu.VMEM((2, CHUNK), jnp.int32),
        pltpu.VMEM((2, CHUNK // 8, 128), jnp.float32),
        pltpu.VMEM((2, CHUNK // 8, 128), jnp.float32),
        pltpu.VMEM((F,), jnp.float32),
        pltpu.VMEM((2, CHUNK, F), jnp.float32),
        pltpu.SemaphoreType.DMA, pltpu.SemaphoreType.DMA,
        pltpu.SemaphoreType.DMA, pltpu.SemaphoreType.DMA,
        pltpu.SemaphoreType.DMA, pltpu.SemaphoreType.DMA,
    ]


def _sc_pass(gather_issue, idxi_hbm, idxj_hbm, phi_hbm, sph_hbm, b_hbm,
             zeros_hbm, out_hbm, acc, idxi_v, idxj_v, sidx_v, phi_v, sph_v,
             b_v, m_v, sems, per_edge):
    c_ax = lax.axis_index("c")
    s_ax = lax.axis_index("s")
    wid = s_ax * 2 + c_ax
    isem, gsem, ssem = sems[0:2], sems[2:4], sems[4:6]
    pltpu.sync_copy(zeros_hbm,
                    acc.at[pl.ds(s_ax * ROWS_PER_TILE, ROWS_PER_TILE)])
    pltpu.sync_copy(b_hbm, b_v)
    bvs = [b_v[pl.ds(k * 16, 16)] for k in range(NSPH)]
    plsc.subcore_barrier()
    tile_base = wid * EDGES_PER_TILE

    def in_issue(c, b):
        base = tile_base + c * CHUNK
        rbase = (tile_base // 8) + c * (CHUNK // 8)
        pltpu.async_copy(idxi_hbm.at[pl.ds(base, CHUNK)], idxi_v.at[b],
                         isem[b])
        pltpu.async_copy(idxj_hbm.at[pl.ds(base, CHUNK)], idxj_v.at[b],
                         isem[b])
        pltpu.async_copy(phi_hbm.at[pl.ds(rbase, CHUNK // 8)], phi_v.at[b],
                         isem[b])
        pltpu.async_copy(sph_hbm.at[pl.ds(rbase, CHUNK // 8)], sph_v.at[b],
                         isem[b])

    def in_wait(b):
        pltpu.make_async_copy(idxi_hbm.at[pl.ds(0, CHUNK)], idxi_v.at[b],
                              isem[b]).wait()
        pltpu.make_async_copy(idxj_hbm.at[pl.ds(0, CHUNK)], idxj_v.at[b],
                              isem[b]).wait()
        pltpu.make_async_copy(phi_hbm.at[pl.ds(0, CHUNK // 8)], phi_v.at[b],
                              isem[b]).wait()
        pltpu.make_async_copy(sph_hbm.at[pl.ds(0, CHUNK // 8)], sph_v.at[b],
                              isem[b]).wait()

    def g_issue(b):
        gather_issue(b, idxj_v.at[b], m_v.at[b], gsem[b], issue=True)

    def g_wait(b):
        gather_issue(b, idxj_v.at[b], m_v.at[b], gsem[b], issue=False)

    def s_issue(b):
        for t in range(CHUNK // 16):
            sidx_v[b, pl.ds(t * 16, 16)] = idxi_v[b, pl.ds(t * 16, 16)]
        pltpu.async_copy(m_v.at[b], acc.at[sidx_v.at[b]], ssem[b], add=True)

    def s_wait(b):
        pltpu.make_async_copy(m_v.at[b], acc.at[sidx_v.at[b]],
                              ssem[b]).wait()

    def cmp(b):
        def _body(e, carry):
            per_edge(e, b, phi_v, sph_v, bvs, m_v)
            return carry
        lax.fori_loop(0, CHUNK, _body, 0)

    def step(c, b, first, has_next, has_next2):
        g_wait(b)
        if not first:
            s_wait(1 - b)
        if has_next:
            in_wait(1 - b)
            g_issue(1 - b)
        cmp(b)
        s_issue(b)
        if has_next2:
            in_issue(c + 2, b)

    in_issue(0, 0)
    in_wait(0)
    g_issue(0)
    in_issue(1, 1)
    step(0, 0, True, True, True)
    step(1, 1, False, True, True)

    def pair(g, carry):
        step(g * 2, 0, False, True, True)
        step(g * 2 + 1, 1, False, True, True)
        return carry
    lax.fori_loop(1, NPAIR - 1, pair, 0)
    step(NCHUNK - 2, 0, False, True, False)
    step(NCHUNK - 1, 1, False, False, False)
    s_wait(1)

    plsc.subcore_barrier()
    row0 = s_ax * ROWS_PER_TILE
    pltpu.sync_copy(acc.at[pl.ds(row0, ROWS_PER_TILE)],
                    out_hbm.at[pl.ds(c_ax * N_PAD + row0, ROWS_PER_TILE)])


@functools.lru_cache(maxsize=None)
def _make_sc_pass1():
    mesh = plsc.VectorSubcoreMesh(core_axis_name="c", subcore_axis_name="s")

    @functools.partial(
        pl.kernel, mesh=mesh,
        out_type=jax.ShapeDtypeStruct((2 * N_PAD, F), jnp.float32),
        scratch_types=_sc_scratch() + [pltpu.VMEM((2, CHUNK, 16),
                                                  jnp.float32)],
        compiler_params=pltpu.CompilerParams(use_tc_tiling_on_sc=False),
    )
    def _sc_pass1(idxi_hbm, idxj_hbm, phi_hbm, sph_hbm, ne_hbm, b1_hbm,
                  zeros_hbm, out_hbm, acc, idxi_v, idxj_v, sidx_v, phi_v,
                  sph_v, b_v, m_v, is0, is1, gs0, gs1, ss0, ss1, ne_v):
        def gather_rows(b, idxj_slot, m_slot, sem, issue):
            cp = pltpu.make_async_copy(ne_hbm.at[idxj_slot], ne_v.at[b], sem)
            if issue:
                cp.start()
            else:
                cp.wait()

        def per_edge(e, b, phi_ref, sph_ref, bvs, m_ref):
            r = e >> 3
            col = (e & 7) * 16
            phi = phi_ref[b, r, pl.ds(col, 16)]
            sphv = sph_ref[b, r, pl.ds(col, 16)]
            ne = ne_v[b, e, :]
            ye = phi * ne
            u = phi * ye
            for k in range(NSPH):
                m = _lane_bcast(sphv, k) * u + bvs[k] * ye
                m_ref[b, e, pl.ds(k * 16, 16)] = m

        _sc_pass(gather_rows, idxi_hbm, idxj_hbm, phi_hbm, sph_hbm, b1_hbm,
                 zeros_hbm, out_hbm, acc, idxi_v, idxj_v, sidx_v, phi_v,
                 sph_v, b_v, m_v, [is0, is1, gs0, gs1, ss0, ss1], per_edge)

    return _sc_pass1


@functools.lru_cache(maxsize=None)
def _make_sc_pass2():
    mesh = plsc.VectorSubcoreMesh(core_axis_name="c", subcore_axis_name="s")

    @functools.partial(
        pl.kernel, mesh=mesh,
        out_type=jax.ShapeDtypeStruct((2 * N_PAD, F), jnp.float32),
        scratch_types=_sc_scratch(),
        compiler_params=pltpu.CompilerParams(use_tc_tiling_on_sc=False),
    )
    def _sc_pass2(idxi_hbm, idxj_hbm, phi_hbm, sph_hbm, y_hbm, b2_hbm,
                  zeros_hbm, out_hbm, acc, idxi_v, idxj_v, sidx_v, phi_v,
                  sph_v, b_v, m_v, is0, is1, gs0, gs1, ss0, ss1):
        def gather_rows(b, idxj_slot, m_slot, sem, issue):
            cp = pltpu.make_async_copy(y_hbm.at[idxj_slot], m_slot, sem)
            if issue:
                cp.start()
            else:
                cp.wait()

        def per_edge(e, b, phi_ref, sph_ref, bvs, m_ref):
            r = e >> 3
            col = (e & 7) * 16
            phi = phi_ref[b, r, pl.ds(col, 16)]
            sphv = sph_ref[b, r, pl.ds(col, 16)]
            for k in range(NSPH):
                t = _lane_bcast(sphv, k) * phi + bvs[k]
                m_ref[b, e, pl.ds(k * 16, 16)] = (
                    t * m_ref[b, e, pl.ds(k * 16, 16)])

        _sc_pass(gather_rows, idxi_hbm, idxj_hbm, phi_hbm, sph_hbm, b2_hbm,
                 zeros_hbm, out_hbm, acc, idxi_v, idxj_v, sidx_v, phi_v,
                 sph_v, b_v, m_v, [is0, is1, gs0, gs1, ss0, ss1], per_edge)

    return _sc_pass2


def kernel(atomic_numbers, neighbour_indices, neighbour_displacements,
           embedding_table, embed_W, embed_b, dense1_W, dense1_b, dense2_W,
           dense2_b, basis_b1, basis_b2, neighbour_normalization):
    f32 = jnp.float32
    pad_e = E_PAD - N_EDGES
    dummy = (N_ATOMS + (jnp.arange(pad_e, dtype=jnp.int32) % 16)).astype(jnp.int32)
    idx = neighbour_indices.astype(jnp.int32)
    idx_i = jnp.concatenate([idx[:, 0], dummy])
    idx_j = jnp.concatenate([idx[:, 1], dummy])
    disp_pad = jnp.concatenate(
        [neighbour_displacements.astype(f32),
         jnp.tile(jnp.asarray([[2.0 * RCUT, 0.0, 0.0]], f32), (pad_e, 1))])
    disp8 = disp_pad.reshape(E_PAD // 8, 8, 3).transpose(1, 2, 0).reshape(
        24, E_PAD // 8)

    z_pad = jnp.concatenate(
        [atomic_numbers.astype(jnp.int32),
         jnp.zeros((N_PAD - N_ATOMS,), jnp.int32)]).reshape(N_PAD, 1)

    eye9 = jnp.eye(NSPH, dtype=f32)
    W144_1 = jnp.kron(eye9, dense1_W.astype(f32))
    W144_2 = jnp.kron(eye9, dense2_W.astype(f32))
    bias1 = jnp.concatenate([dense1_b.astype(f32),
                             jnp.zeros((F - 16,), f32)]).reshape(1, F)
    bias2 = jnp.concatenate([dense2_b.astype(f32),
                             jnp.zeros((F - 16,), f32)]).reshape(1, F)
    invn = (1.0 / neighbour_normalization.astype(f32)).reshape(1, 1)
    zeros_stripe = jnp.zeros((ROWS_PER_TILE, F), f32)
    zero_extra = jnp.zeros((1, F), f32)

    phi_arr, sph_arr = _edge_features(disp8)
    ne, y0 = _node_embed(z_pad, embedding_table.astype(f32),
                         embed_W.astype(f32), embed_b.astype(f32).reshape(1, 16))

    acc1 = _make_sc_pass1()(idx_i, idx_j, phi_arr, sph_arr, ne,
                            basis_b1.astype(f32).reshape(F), zeros_stripe)
    y_mid = _dense(acc1, W144_1, bias1, invn, zero_extra)
    acc2 = _make_sc_pass2()(idx_i, idx_j, phi_arr, sph_arr, y_mid,
                            basis_b2.astype(f32).reshape(F), zeros_stripe)
    y0_ext = jnp.concatenate([y0, jnp.zeros((N_PAD, F - 16), f32)], axis=1)
    y_out = _dense(acc2, W144_2, bias2, invn, y0_ext)

    return y_out[:N_ATOMS].reshape(N_ATOMS, NSPH, NRAD)


# speedup vs baseline: 1.4388x; 1.0846x over previous
import functools

import jax
import jax.numpy as jnp
from jax import lax
from jax.experimental import pallas as pl
from jax.experimental.pallas import tpu as pltpu
from jax.experimental.pallas import tpu_sc as plsc

N_ATOMS = 10000
N_EDGES = 320000
NRAD = 16
NSPEC = 100
NSPH = 9
RCUT = 5.0

N_PAD = 10016
E_PAD = 323584
EDGES_PER_TILE = E_PAD // 32
CHUNK = 64
NCHUNK = EDGES_PER_TILE // CHUNK
NPAIR = NCHUNK // 2
ROWS_PER_TILE = N_PAD // 16
F = NSPH * NRAD
BE = 2048


def _t128(a):
    eye = jnp.eye(128, dtype=jnp.float32)
    return lax.dot_general(a, eye, (((0,), (0,)), ((), ())),
                           preferred_element_type=jnp.float32)


def _edge_feat_body(disp8_ref, phi_ref, sph_ref):
    B8 = BE // 8
    d = disp8_ref[...]
    k = (lax.broadcasted_iota(jnp.int32, (16, B8), 0) + 1).astype(jnp.float32)
    invk = 1.0 / k
    phis, sphs = [], []
    for a in range(8):
        x = d[3 * a:3 * a + 1, :]
        y = d[3 * a + 1:3 * a + 2, :]
        z = d[3 * a + 2:3 * a + 3, :]
        r2 = x * x + y * y + z * z + 1e-12
        invr = lax.rsqrt(r2)
        r = r2 * invr
        xh, yh, zh = x * invr, y * invr, z * invr
        theta = (jnp.pi / RCUT) * r
        cut = jnp.where(r < RCUT, 0.5 * (jnp.cos(theta) + 1.0), 0.0)
        xarg = k * theta
        phis.append(jnp.sin(xarg) * invk * (cut / theta))
        zeros7 = jnp.zeros((7, B8), jnp.float32)
        sphs.append(jnp.concatenate(
            [jnp.ones_like(x), yh, zh, xh, xh * yh, yh * zh,
             3.0 * zh * zh - 1.0, xh * zh, xh * xh - yh * yh, zeros7],
            axis=0))
    phi_ref[...] = _t128(jnp.concatenate(phis, axis=0))
    sph_ref[...] = _t128(jnp.concatenate(sphs, axis=0))


def _edge_features(disp8):
    grid = E_PAD // BE
    return pl.pallas_call(
        _edge_feat_body,
        grid=(grid,),
        in_specs=[pl.BlockSpec((24, BE // 8), lambda i: (0, i))],
        out_specs=[pl.BlockSpec((BE // 8, 128), lambda i: (i, 0)),
                   pl.BlockSpec((BE // 8, 128), lambda i: (i, 0))],
        out_shape=[jax.ShapeDtypeStruct((E_PAD // 8, 128), jnp.float32),
                   jax.ShapeDtypeStruct((E_PAD // 8, 128), jnp.float32)],
    )(disp8)


def _node_embed_body(z_ref, emb_ref, eW_ref, eb_ref, ne_ref, y0_ref):
    z = z_ref[...]
    iot = lax.broadcasted_iota(jnp.int32, (N_PAD, NSPEC), 1)
    oh = (z == iot).astype(jnp.float32)
    ne = jnp.dot(oh, emb_ref[...], preferred_element_type=jnp.float32)
    ne_ref[...] = ne
    y0_ref[...] = jnp.dot(ne, eW_ref[...],
                          preferred_element_type=jnp.float32) + eb_ref[...]


def _node_embed(z_pad, emb, eW, eb):
    return pl.pallas_call(
        _node_embed_body,
        out_shape=[jax.ShapeDtypeStruct((N_PAD, 16), jnp.float32),
                   jax.ShapeDtypeStruct((N_PAD, 16), jnp.float32)],
    )(z_pad, emb, eW, eb)


def _dense_body(acc_ref, W_ref, bias_ref, invn_ref, extra_ref, y_ref):
    a = acc_ref[0:N_PAD, :] + acc_ref[N_PAD:2 * N_PAD, :]
    y1 = a * invn_ref[0, 0]
    h = jnp.dot(y1, W_ref[...], preferred_element_type=jnp.float32)
    y_ref[...] = h + y1 + bias_ref[...] + extra_ref[...]


def _dense(acc, W144, bias_row, invn, extra):
    return pl.pallas_call(
        _dense_body,
        out_shape=jax.ShapeDtypeStruct((N_PAD, F), jnp.float32),
    )(acc, W144, bias_row, invn, extra)


def _lane_bcast(v, s):
    idx = jnp.full((16, 1), s, jnp.int32)
    dn = lax.GatherDimensionNumbers(offset_dims=(), collapsed_slice_dims=(0,),
                                    start_index_map=(0,))
    return lax.gather(v, idx, dn, (1,),
                      mode=lax.GatherScatterMode.PROMISE_IN_BOUNDS)


def _sc_scratch():
    return [
        pltpu.VMEM_SHARED((N_PAD, F), jnp.float32),
        pltpu.VMEM((2, CHUNK), jnp.int32),
        pltpu.VMEM((2, CHUNK), jnp.int32),
        pltpu.VMEM((2, CHUNK), jnp.int32),
        pltpu.VMEM((2, CHUNK // 8, 128), jnp.float32),
        pltpu.VMEM((2, CHUNK // 8, 128), jnp.float32),
        pltpu.VMEM((F,), jnp.float32),
        pltpu.VMEM((2, CHUNK, F), jnp.float32),
        pltpu.SemaphoreType.DMA, pltpu.SemaphoreType.DMA,
        pltpu.SemaphoreType.DMA, pltpu.SemaphoreType.DMA,
        pltpu.SemaphoreType.DMA, pltpu.SemaphoreType.DMA,
    ]


def _sc_pass(gather_issue, idxi_hbm, idxj_hbm, phi_hbm, sph_hbm, b_hbm,
             zeros_hbm, out_hbm, acc, idxi_v, idxj_v, sidx_v, phi_v, sph_v,
             b_v, m_v, sems, per_edge):
    c_ax = lax.axis_index("c")
    s_ax = lax.axis_index("s")
    wid = s_ax * 2 + c_ax
    isem, gsem, ssem = sems[0:2], sems[2:4], sems[4:6]
    pltpu.sync_copy(zeros_hbm,
                    acc.at[pl.ds(s_ax * ROWS_PER_TILE, ROWS_PER_TILE)])
    pltpu.sync_copy(b_hbm, b_v)
    bvs = [b_v[pl.ds(k * 16, 16)] for k in range(NSPH)]
    plsc.subcore_barrier()
    tile_base = wid * EDGES_PER_TILE

    def in_issue(c, b):
        base = tile_base + c * CHUNK
        rbase = (tile_base // 8) + c * (CHUNK // 8)
        pltpu.async_copy(idxi_hbm.at[pl.ds(base, CHUNK)], idxi_v.at[b],
                         isem[b])
        pltpu.async_copy(idxj_hbm.at[pl.ds(base, CHUNK)], idxj_v.at[b],
                         isem[b])
        pltpu.async_copy(phi_hbm.at[pl.ds(rbase, CHUNK // 8)], phi_v.at[b],
                         isem[b])
        pltpu.async_copy(sph_hbm.at[pl.ds(rbase, CHUNK // 8)], sph_v.at[b],
                         isem[b])

    def in_wait(b):
        pltpu.make_async_copy(idxi_hbm.at[pl.ds(0, CHUNK)], idxi_v.at[b],
                              isem[b]).wait()
        pltpu.make_async_copy(idxj_hbm.at[pl.ds(0, CHUNK)], idxj_v.at[b],
                              isem[b]).wait()
        pltpu.make_async_copy(phi_hbm.at[pl.ds(0, CHUNK // 8)], phi_v.at[b],
                              isem[b]).wait()
        pltpu.make_async_copy(sph_hbm.at[pl.ds(0, CHUNK // 8)], sph_v.at[b],
                              isem[b]).wait()

    def g_issue(b):
        gather_issue(b, idxj_v.at[b], m_v.at[b], gsem[b], issue=True)

    def g_wait(b):
        gather_issue(b, idxj_v.at[b], m_v.at[b], gsem[b], issue=False)

    def s_issue(b):
        for t in range(CHUNK // 16):
            sidx_v[b, pl.ds(t * 16, 16)] = idxi_v[b, pl.ds(t * 16, 16)]
        pltpu.async_copy(m_v.at[b], acc.at[sidx_v.at[b]], ssem[b], add=True)

    def s_wait(b):
        pltpu.make_async_copy(m_v.at[b], acc.at[sidx_v.at[b]],
                              ssem[b]).wait()

    def load_edge(e, b):
        r = e >> 3
        col = (e & 7) * 16
        return (phi_v[b, r, pl.ds(col, 16)], sph_v[b, r, pl.ds(col, 16)])

    def cmp(b):
        def _body(e, carry):
            en = (e + 1) & (CHUNK - 1)
            nxt = load_edge(en, b)
            per_edge(e, b, carry, bvs, m_v)
            return nxt
        lax.fori_loop(0, CHUNK, _body, load_edge(0, b))

    def step(c, b, first, has_next, has_next2):
        g_wait(b)
        if not first:
            s_wait(1 - b)
        if has_next:
            in_wait(1 - b)
            g_issue(1 - b)
        cmp(b)
        s_issue(b)
        if has_next2:
            in_issue(c + 2, b)

    in_issue(0, 0)
    in_wait(0)
    g_issue(0)
    in_issue(1, 1)
    step(0, 0, True, True, True)
    step(1, 1, False, True, True)

    def pair(g, carry):
        step(g * 2, 0, False, True, True)
        step(g * 2 + 1, 1, False, True, True)
        return carry
    lax.fori_loop(1, NPAIR - 1, pair, 0)
    step(NCHUNK - 2, 0, False, True, False)
    step(NCHUNK - 1, 1, False, False, False)
    s_wait(1)

    plsc.subcore_barrier()
    row0 = s_ax * ROWS_PER_TILE
    pltpu.sync_copy(acc.at[pl.ds(row0, ROWS_PER_TILE)],
                    out_hbm.at[pl.ds(c_ax * N_PAD + row0, ROWS_PER_TILE)])


@functools.lru_cache(maxsize=None)
def _make_sc_pass1():
    mesh = plsc.VectorSubcoreMesh(core_axis_name="c", subcore_axis_name="s")

    @functools.partial(
        pl.kernel, mesh=mesh,
        out_type=jax.ShapeDtypeStruct((2 * N_PAD, F), jnp.float32),
        scratch_types=_sc_scratch() + [pltpu.VMEM((2, CHUNK, 16),
                                                  jnp.float32)],
        compiler_params=pltpu.CompilerParams(use_tc_tiling_on_sc=False),
    )
    def _sc_pass1(idxi_hbm, idxj_hbm, phi_hbm, sph_hbm, ne_hbm, b1_hbm,
                  zeros_hbm, out_hbm, acc, idxi_v, idxj_v, sidx_v, phi_v,
                  sph_v, b_v, m_v, is0, is1, gs0, gs1, ss0, ss1, ne_v):
        def gather_rows(b, idxj_slot, m_slot, sem, issue):
            cp = pltpu.make_async_copy(ne_hbm.at[idxj_slot], ne_v.at[b], sem)
            if issue:
                cp.start()
            else:
                cp.wait()

        def per_edge(e, b, feat, bvs, m_ref):
            phi, sphv = feat
            ne = ne_v[b, e, :]
            ye = phi * ne
            u = phi * ye
            for k in range(NSPH):
                m = _lane_bcast(sphv, k) * u + bvs[k] * ye
                m_ref[b, e, pl.ds(k * 16, 16)] = m

        _sc_pass(gather_rows, idxi_hbm, idxj_hbm, phi_hbm, sph_hbm, b1_hbm,
                 zeros_hbm, out_hbm, acc, idxi_v, idxj_v, sidx_v, phi_v,
                 sph_v, b_v, m_v, [is0, is1, gs0, gs1, ss0, ss1], per_edge)

    return _sc_pass1


@functools.lru_cache(maxsize=None)
def _make_sc_pass2():
    mesh = plsc.VectorSubcoreMesh(core_axis_name="c", subcore_axis_name="s")

    @functools.partial(
        pl.kernel, mesh=mesh,
        out_type=jax.ShapeDtypeStruct((2 * N_PAD, F), jnp.float32),
        scratch_types=_sc_scratch(),
        compiler_params=pltpu.CompilerParams(use_tc_tiling_on_sc=False),
    )
    def _sc_pass2(idxi_hbm, idxj_hbm, phi_hbm, sph_hbm, y_hbm, b2_hbm,
                  zeros_hbm, out_hbm, acc, idxi_v, idxj_v, sidx_v, phi_v,
                  sph_v, b_v, m_v, is0, is1, gs0, gs1, ss0, ss1):
        def gather_rows(b, idxj_slot, m_slot, sem, issue):
            cp = pltpu.make_async_copy(y_hbm.at[idxj_slot], m_slot, sem)
            if issue:
                cp.start()
            else:
                cp.wait()

        def per_edge(e, b, feat, bvs, m_ref):
            phi, sphv = feat
            for k in range(NSPH):
                t = _lane_bcast(sphv, k) * phi + bvs[k]
                m_ref[b, e, pl.ds(k * 16, 16)] = (
                    t * m_ref[b, e, pl.ds(k * 16, 16)])

        _sc_pass(gather_rows, idxi_hbm, idxj_hbm, phi_hbm, sph_hbm, b2_hbm,
                 zeros_hbm, out_hbm, acc, idxi_v, idxj_v, sidx_v, phi_v,
                 sph_v, b_v, m_v, [is0, is1, gs0, gs1, ss0, ss1], per_edge)

    return _sc_pass2


def kernel(atomic_numbers, neighbour_indices, neighbour_displacements,
           embedding_table, embed_W, embed_b, dense1_W, dense1_b, dense2_W,
           dense2_b, basis_b1, basis_b2, neighbour_normalization):
    f32 = jnp.float32
    pad_e = E_PAD - N_EDGES
    dummy = (N_ATOMS + (jnp.arange(pad_e, dtype=jnp.int32) % 16)).astype(jnp.int32)
    idx = neighbour_indices.astype(jnp.int32)
    idx_i = jnp.concatenate([idx[:, 0], dummy])
    idx_j = jnp.concatenate([idx[:, 1], dummy])
    disp_pad = jnp.concatenate(
        [neighbour_displacements.astype(f32),
         jnp.tile(jnp.asarray([[2.0 * RCUT, 0.0, 0.0]], f32), (pad_e, 1))])
    disp8 = disp_pad.reshape(E_PAD // 8, 8, 3).transpose(1, 2, 0).reshape(
        24, E_PAD // 8)

    z_pad = jnp.concatenate(
        [atomic_numbers.astype(jnp.int32),
         jnp.zeros((N_PAD - N_ATOMS,), jnp.int32)]).reshape(N_PAD, 1)

    eye9 = jnp.eye(NSPH, dtype=f32)
    W144_1 = jnp.kron(eye9, dense1_W.astype(f32))
    W144_2 = jnp.kron(eye9, dense2_W.astype(f32))
    bias1 = jnp.concatenate([dense1_b.astype(f32),
                             jnp.zeros((F - 16,), f32)]).reshape(1, F)
    bias2 = jnp.concatenate([dense2_b.astype(f32),
                             jnp.zeros((F - 16,), f32)]).reshape(1, F)
    invn = (1.0 / neighbour_normalization.astype(f32)).reshape(1, 1)
    zeros_stripe = jnp.zeros((ROWS_PER_TILE, F), f32)
    zero_extra = jnp.zeros((1, F), f32)

    phi_arr, sph_arr = _edge_features(disp8)
    ne, y0 = _node_embed(z_pad, embedding_table.astype(f32),
                         embed_W.astype(f32), embed_b.astype(f32).reshape(1, 16))

    acc1 = _make_sc_pass1()(idx_i, idx_j, phi_arr, sph_arr, ne,
                            basis_b1.astype(f32).reshape(F), zeros_stripe)
    y_mid = _dense(acc1, W144_1, bias1, invn, zero_extra)
    acc2 = _make_sc_pass2()(idx_i, idx_j, phi_arr, sph_arr, y_mid,
                            basis_b2.astype(f32).reshape(F), zeros_stripe)
    y0_ext = jnp.concatenate([y0, jnp.zeros((N_PAD, F - 16), f32)], axis=1)
    y_out = _dense(acc2, W144_2, bias2, invn, y0_ext)

    return y_out[:N_ATOMS].reshape(N_ATOMS, NSPH, NRAD)
